# baseline (speedup 1.0000x reference)
import functools
import jax
import jax.numpy as jnp
from jax.experimental import pallas as pl
from jax.experimental.pallas import tpu as pltpu

N = 8192
K = 20
BN_EPS = 1e-5
NEG = -jnp.inf


def _leaky(y):
    return jnp.where(y >= 0, y, 0.2 * y)


def _act(y):
    return _leaky(y / jnp.sqrt(1.0 + BN_EPS))


CHUNK = 128
NCHUNK = N // CHUNK
DEPTH = 6


def _naive_topk(D):
    TM = D.shape[0]
    iota = jax.lax.broadcasted_iota(jnp.int32, (TM, N), 1)
    cols = []
    for _ in range(K):
        m = jnp.max(D, axis=1, keepdims=True)
        sel = jnp.min(jnp.where(D == m, iota, N), axis=1, keepdims=True)
        cols.append(sel)
        D = jnp.where(iota == sel, NEG, D)
    return jnp.concatenate(cols, axis=1)


def _chunked_topk(D):
    TM = D.shape[0]
    Dr = D.reshape(TM, NCHUNK, CHUNK)
    iota_s = jax.lax.broadcasted_iota(jnp.int32, (TM, NCHUNK, CHUNK), 2)
    Mv = []
    Pv = []
    m = jnp.max(Dr, axis=2)
    p = jnp.min(jnp.where(Dr == m[:, :, None], iota_s, CHUNK), axis=2)
    Mv.append(m)
    Pv.append(p)
    for _ in range(1, DEPTH):
        mp = Mv[-1][:, :, None]
        pp = Pv[-1][:, :, None]
        keep = (Dr < mp) | ((Dr == mp) & (iota_s > pp))
        m = jnp.max(jnp.where(keep, Dr, NEG), axis=2)
        p = jnp.min(jnp.where((Dr == m[:, :, None]) & keep, iota_s, CHUNK),
                    axis=2)
        Mv.append(m)
        Pv.append(p)

    iota_g = jax.lax.broadcasted_iota(jnp.int32, (TM, NCHUNK), 1)
    H = Mv[0]
    HP = Pv[0]
    CNT = jnp.ones((TM, NCHUNK), jnp.int32)
    bad = jnp.zeros((TM, 1), jnp.bool_)
    cols = []
    for _ in range(K):
        m = jnp.max(H, axis=1, keepdims=True)
        gsel = jnp.min(jnp.where(H == m, iota_g, NCHUNK), axis=1,
                       keepdims=True)
        hit = iota_g == gsel
        pos = jnp.sum(jnp.where(hit, HP, 0), axis=1, keepdims=True)
        cols.append(gsel * CHUNK + pos)
        nh = NEG * jnp.ones_like(H)
        np_ = jnp.full_like(HP, CHUNK)
        for d in range(1, DEPTH):
            sel_d = CNT == d
            nh = jnp.where(sel_d, Mv[d], nh)
            np_ = jnp.where(sel_d, Pv[d], np_)
        exhausted = CNT >= DEPTH
        bad = bad | jnp.any(hit & exhausted, axis=1, keepdims=True)
        H = jnp.where(hit, nh, H)
        HP = jnp.where(hit, np_, HP)
        CNT = jnp.where(hit, CNT + 1, CNT)
    fast = jnp.concatenate(cols, axis=1)
    return jax.lax.cond(jnp.any(bad), lambda: _naive_topk(D), lambda: fast)


def _knn_body(xrows_ref, xall_ref, ncol_ref, nrow_ref, idx_ref):
    xr = xrows_ref[...]
    xa = xall_ref[...]
    inner = jax.lax.dot_general(xr, xa, (((1,), (1,)), ((), ())),
                                preferred_element_type=jnp.float32,
                                precision=jax.lax.Precision.DEFAULT)
    D = (2.0 * inner - ncol_ref[...]) - nrow_ref[...]
    idx_ref[...] = _chunked_topk(D)


@functools.partial(jax.jit, static_argnames=("tm",))
def _knn_idx(xT, nrm, tm=128):
    C = xT.shape[1]
    grid = (N // tm,)
    return pl.pallas_call(
        _knn_body,
        grid=grid,
        compiler_params=pltpu.CompilerParams(
            vmem_limit_bytes=100 * 1024 * 1024),
        in_specs=[
            pl.BlockSpec((tm, C), lambda i: (i, 0)),
            pl.BlockSpec((N, C), lambda i: (0, 0)),
            pl.BlockSpec((1, N), lambda i: (0, 0)),
            pl.BlockSpec((tm, 1), lambda i: (i, 0)),
        ],
        out_specs=pl.BlockSpec((tm, K), lambda i: (i, 0)),
        out_shape=jax.ShapeDtypeStruct((N, K), jnp.int32),
    )(xT, xT, nrm[None, :], nrm[:, None])


def _conv_body(feat_ref, w_ref, out_ref):
    f = feat_ref[...]
    w = w_ref[...]
    y = jax.lax.dot_general(f, w, (((1,), (1,)), ((), ())),
                            preferred_element_type=jnp.float32,
                            precision=jax.lax.Precision.DEFAULT)
    tm = y.shape[0] // K
    m = jnp.max(y.reshape(tm, K, y.shape[1]), axis=1)
    out_ref[...] = _act(m)


@functools.partial(jax.jit, static_argnames=("tm",))
def _conv_max(feat, W, tm=128):
    O = W.shape[0]
    C2 = W.shape[1]
    grid = (N // tm,)
    return pl.pallas_call(
        _conv_body,
        grid=grid,
        in_specs=[
            pl.BlockSpec((tm * K, C2), lambda i: (i, 0)),
            pl.BlockSpec((O, C2), lambda i: (0, 0)),
        ],
        out_specs=pl.BlockSpec((tm, O), lambda i: (i, 0)),
        out_shape=jax.ShapeDtypeStruct((N, O), jnp.float32),
    )(feat, W)


def _edge_layer(xT, W):
    C = xT.shape[1]
    pad = (-C) % 8
    xp = jnp.pad(xT, ((0, 0), (0, pad))) if pad else xT
    nrm = jnp.sum(xT * xT, axis=1)
    idx = _knn_idx(xp, nrm)
    nb = jnp.take(xT, idx.reshape(-1), axis=0)
    center = jnp.repeat(xT, K, axis=0)
    feat = jnp.concatenate([nb - center, center], axis=1)
    return _conv_max(feat, W)


def kernel(x, W1, W2, W3, W4, W5):
    xT = x[0].T
    x1 = _edge_layer(xT, W1)
    x2 = _edge_layer(x1, W2)
    x3 = _edge_layer(x2, W3)
    x4 = _edge_layer(x3, W4)
    cat = jnp.concatenate([x1, x2, x3, x4], axis=1)
    x5 = _act(cat @ W5.T)
    xg = jnp.max(x5, axis=0)
    x5t = x5.T
    out = jnp.concatenate(
        [x5t, jnp.broadcast_to(xg[:, None], x5t.shape)], axis=0)
    return out[None]


# speedup vs baseline: 1.6010x; 1.6010x over previous
import functools
import jax
import jax.numpy as jnp
from jax.experimental import pallas as pl
from jax.experimental.pallas import tpu as pltpu

N = 8192
K = 20
BN_EPS = 1e-5
NEG = -jnp.inf


def _leaky(y):
    return jnp.where(y >= 0, y, 0.2 * y)


def _act(y):
    return _leaky(y / jnp.sqrt(1.0 + BN_EPS))


CHUNK = 128
NCHUNK = N // CHUNK
DEPTH = 6


def _naive_topk(D):
    TM = D.shape[0]
    iota = jax.lax.broadcasted_iota(jnp.int32, (TM, N), 1)
    cols = []
    for _ in range(K):
        m = jnp.max(D, axis=1, keepdims=True)
        sel = jnp.min(jnp.where(D == m, iota, N), axis=1, keepdims=True)
        cols.append(sel)
        D = jnp.where(iota == sel, NEG, D)
    return jnp.concatenate(cols, axis=1)


def _chunked_topk(D):
    TM = D.shape[0]
    Dr = D.reshape(TM, NCHUNK, CHUNK)
    iota_s = jax.lax.broadcasted_iota(jnp.int32, (TM, NCHUNK, CHUNK), 2)
    Mv = []
    Pv = []
    m = jnp.max(Dr, axis=2)
    p = jnp.min(jnp.where(Dr == m[:, :, None], iota_s, CHUNK), axis=2)
    Mv.append(m)
    Pv.append(p)
    for _ in range(1, DEPTH):
        mp = Mv[-1][:, :, None]
        pp = Pv[-1][:, :, None]
        keep = (Dr < mp) | ((Dr == mp) & (iota_s > pp))
        m = jnp.max(jnp.where(keep, Dr, NEG), axis=2)
        p = jnp.min(jnp.where((Dr == m[:, :, None]) & keep, iota_s, CHUNK),
                    axis=2)
        Mv.append(m)
        Pv.append(p)

    iota_g = jax.lax.broadcasted_iota(jnp.int32, (TM, NCHUNK), 1)
    H = Mv[0]
    HP = Pv[0]
    CNT = jnp.ones((TM, NCHUNK), jnp.int32)
    bad = jnp.zeros((TM, 1), jnp.bool_)
    cols = []
    for _ in range(K):
        m = jnp.max(H, axis=1, keepdims=True)
        gsel = jnp.min(jnp.where(H == m, iota_g, NCHUNK), axis=1,
                       keepdims=True)
        hit = iota_g == gsel
        pos = jnp.sum(jnp.where(hit, HP, 0), axis=1, keepdims=True)
        cols.append(gsel * CHUNK + pos)
        nh = NEG * jnp.ones_like(H)
        np_ = jnp.full_like(HP, CHUNK)
        for d in range(1, DEPTH):
            sel_d = CNT == d
            nh = jnp.where(sel_d, Mv[d], nh)
            np_ = jnp.where(sel_d, Pv[d], np_)
        exhausted = CNT >= DEPTH
        bad = bad | jnp.any(hit & exhausted, axis=1, keepdims=True)
        H = jnp.where(hit, nh, H)
        HP = jnp.where(hit, np_, HP)
        CNT = jnp.where(hit, CNT + 1, CNT)
    fast = jnp.concatenate(cols, axis=1)
    return jax.lax.cond(jnp.any(bad), lambda: _naive_topk(D), lambda: fast)


def _argmax_topk(D):
    TM = D.shape[0]
    iota = jax.lax.broadcasted_iota(jnp.int32, (TM, N), 1)
    cols = []
    for _ in range(K):
        sel = jnp.argmax(D, axis=1)[:, None]
        cols.append(sel)
        D = jnp.where(iota == sel, NEG, D)
    return jnp.concatenate(cols, axis=1)


def _knn_body(xrows_ref, xall_ref, ncol_ref, nrow_ref, idx_ref):
    xr = xrows_ref[...]
    xa = xall_ref[...]
    inner = jax.lax.dot_general(xr, xa, (((1,), (1,)), ((), ())),
                                preferred_element_type=jnp.float32,
                                precision=jax.lax.Precision.DEFAULT)
    D = (2.0 * inner - ncol_ref[...]) - nrow_ref[...]
    idx_ref[...] = _argmax_topk(D)


@functools.partial(jax.jit, static_argnames=("tm",))
def _knn_idx(xT, nrm, tm=512):
    C = xT.shape[1]
    grid = (N // tm,)
    return pl.pallas_call(
        _knn_body,
        grid=grid,
        compiler_params=pltpu.CompilerParams(
            vmem_limit_bytes=100 * 1024 * 1024),
        in_specs=[
            pl.BlockSpec((tm, C), lambda i: (i, 0)),
            pl.BlockSpec((N, C), lambda i: (0, 0)),
            pl.BlockSpec((1, N), lambda i: (0, 0)),
            pl.BlockSpec((tm, 1), lambda i: (i, 0)),
        ],
        out_specs=pl.BlockSpec((tm, K), lambda i: (i, 0)),
        out_shape=jax.ShapeDtypeStruct((N, K), jnp.int32),
    )(xT, xT, nrm[None, :], nrm[:, None])


def _conv_body(feat_ref, w_ref, out_ref):
    f = feat_ref[...]
    w = w_ref[...]
    y = jax.lax.dot_general(f, w, (((1,), (1,)), ((), ())),
                            preferred_element_type=jnp.float32,
                            precision=jax.lax.Precision.DEFAULT)
    tm = y.shape[0] // K
    m = jnp.max(y.reshape(tm, K, y.shape[1]), axis=1)
    out_ref[...] = _act(m)


@functools.partial(jax.jit, static_argnames=("tm",))
def _conv_max(feat, W, tm=128):
    O = W.shape[0]
    C2 = W.shape[1]
    grid = (N // tm,)
    return pl.pallas_call(
        _conv_body,
        grid=grid,
        in_specs=[
            pl.BlockSpec((tm * K, C2), lambda i: (i, 0)),
            pl.BlockSpec((O, C2), lambda i: (0, 0)),
        ],
        out_specs=pl.BlockSpec((tm, O), lambda i: (i, 0)),
        out_shape=jax.ShapeDtypeStruct((N, O), jnp.float32),
    )(feat, W)


def _edge_layer(xT, W):
    C = xT.shape[1]
    pad = (-C) % 8
    xp = jnp.pad(xT, ((0, 0), (0, pad))) if pad else xT
    nrm = jnp.sum(xT * xT, axis=1)
    idx = _knn_idx(xp, nrm)
    nb = jnp.take(xT, idx.reshape(-1), axis=0)
    center = jnp.repeat(xT, K, axis=0)
    feat = jnp.concatenate([nb - center, center], axis=1)
    return _conv_max(feat, W)


def kernel(x, W1, W2, W3, W4, W5):
    xT = x[0].T
    x1 = _edge_layer(xT, W1)
    x2 = _edge_layer(x1, W2)
    x3 = _edge_layer(x2, W3)
    x4 = _edge_layer(x3, W4)
    cat = jnp.concatenate([x1, x2, x3, x4], axis=1)
    x5 = _act(cat @ W5.T)
    xg = jnp.max(x5, axis=0)
    x5t = x5.T
    out = jnp.concatenate(
        [x5t, jnp.broadcast_to(xg[:, None], x5t.shape)], axis=0)
    return out[None]


# speedup vs baseline: 2.2283x; 1.3918x over previous
import functools
import jax
import jax.numpy as jnp
from jax.experimental import pallas as pl
from jax.experimental.pallas import tpu as pltpu

N = 8192
K = 20
BN_EPS = 1e-5
NEG = -jnp.inf


def _leaky(y):
    return jnp.where(y >= 0, y, 0.2 * y)


def _act(y):
    return _leaky(y / jnp.sqrt(1.0 + BN_EPS))


CHUNK = 128
NCHUNK = N // CHUNK
DEPTH = 6


def _naive_topk(D):
    TM = D.shape[0]
    iota = jax.lax.broadcasted_iota(jnp.int32, (TM, N), 1)
    cols = []
    for _ in range(K):
        m = jnp.max(D, axis=1, keepdims=True)
        sel = jnp.min(jnp.where(D == m, iota, N), axis=1, keepdims=True)
        cols.append(sel)
        D = jnp.where(iota == sel, NEG, D)
    return jnp.concatenate(cols, axis=1)


def _chunked_topk(D):
    TM = D.shape[0]
    Dr = D.reshape(TM, NCHUNK, CHUNK)
    iota_s = jax.lax.broadcasted_iota(jnp.int32, (TM, NCHUNK, CHUNK), 2)
    Mv = []
    Pv = []
    m = jnp.max(Dr, axis=2)
    p = jnp.min(jnp.where(Dr == m[:, :, None], iota_s, CHUNK), axis=2)
    Mv.append(m)
    Pv.append(p)
    for _ in range(1, DEPTH):
        mp = Mv[-1][:, :, None]
        pp = Pv[-1][:, :, None]
        keep = (Dr < mp) | ((Dr == mp) & (iota_s > pp))
        m = jnp.max(jnp.where(keep, Dr, NEG), axis=2)
        p = jnp.min(jnp.where((Dr == m[:, :, None]) & keep, iota_s, CHUNK),
                    axis=2)
        Mv.append(m)
        Pv.append(p)

    iota_g = jax.lax.broadcasted_iota(jnp.int32, (TM, NCHUNK), 1)
    H = Mv[0]
    HP = Pv[0]
    CNT = jnp.ones((TM, NCHUNK), jnp.int32)
    bad = jnp.zeros((TM, 1), jnp.bool_)
    cols = []
    for _ in range(K):
        m = jnp.max(H, axis=1, keepdims=True)
        gsel = jnp.min(jnp.where(H == m, iota_g, NCHUNK), axis=1,
                       keepdims=True)
        hit = iota_g == gsel
        pos = jnp.sum(jnp.where(hit, HP, 0), axis=1, keepdims=True)
        cols.append(gsel * CHUNK + pos)
        nh = NEG * jnp.ones_like(H)
        np_ = jnp.full_like(HP, CHUNK)
        for d in range(1, DEPTH):
            sel_d = CNT == d
            nh = jnp.where(sel_d, Mv[d], nh)
            np_ = jnp.where(sel_d, Pv[d], np_)
        exhausted = CNT >= DEPTH
        bad = bad | jnp.any(hit & exhausted, axis=1, keepdims=True)
        H = jnp.where(hit, nh, H)
        HP = jnp.where(hit, np_, HP)
        CNT = jnp.where(hit, CNT + 1, CNT)
    fast = jnp.concatenate(cols, axis=1)
    return jax.lax.cond(jnp.any(bad), lambda: _naive_topk(D), lambda: fast)


def _argmax_topk(D):
    TM = D.shape[0]
    iota = jax.lax.broadcasted_iota(jnp.int32, (TM, N), 1)
    cols = []
    for _ in range(K):
        sel = jnp.argmax(D, axis=1)[:, None]
        cols.append(sel)
        D = jnp.where(iota == sel, NEG, D)
    return jnp.concatenate(cols, axis=1)


LANES = 128
GSTEPS = N // LANES
FDEPTH = 5


def _file_topk(D):
    TM = D.shape[0]
    vs = [jnp.full((TM, LANES), NEG, jnp.float32) for _ in range(FDEPTH)]
    ps = [jnp.zeros((TM, LANES), jnp.int32) for _ in range(FDEPTH)]
    for g in range(GSTEPS):
        x = jax.lax.slice(D, (0, g * LANES), (TM, (g + 1) * LANES))
        gi = jnp.full((TM, LANES), g, jnp.int32)
        for d in range(FDEPTH):
            c = x > vs[d]
            nv = jnp.maximum(vs[d], x)
            dv = jnp.minimum(vs[d], x)
            np_ = jnp.where(c, gi, ps[d])
            dp = jnp.where(c, ps[d], gi)
            vs[d] = nv
            ps[d] = np_
            x = dv
            gi = dp

    iota_l = jax.lax.broadcasted_iota(jnp.int32, (TM, LANES), 1)
    H = vs[0]
    HP = ps[0]
    CNT = jnp.ones((TM, LANES), jnp.int32)
    bad = jnp.zeros((TM, 1), jnp.bool_)
    cols = []
    for _ in range(K):
        m = jnp.max(H, axis=1, keepdims=True)
        ssel = jnp.min(jnp.where(H == m, iota_l, LANES), axis=1,
                       keepdims=True)
        hit = iota_l == ssel
        gpos = jnp.sum(jnp.where(hit, HP, 0), axis=1, keepdims=True)
        cols.append(gpos * LANES + ssel)
        nh = jnp.full_like(H, NEG)
        nhp = jnp.zeros_like(HP)
        for d in range(1, FDEPTH):
            sel_d = CNT == d
            nh = jnp.where(sel_d, vs[d], nh)
            nhp = jnp.where(sel_d, ps[d], nhp)
        bad = bad | jnp.any(hit & (CNT >= FDEPTH), axis=1, keepdims=True)
        H = jnp.where(hit, nh, H)
        HP = jnp.where(hit, nhp, HP)
        CNT = jnp.where(hit, CNT + 1, CNT)
    fast = jnp.concatenate(cols, axis=1)
    return jax.lax.cond(jnp.any(bad), lambda: _argmax_topk(D), lambda: fast)


def _knn_body(xrows_ref, xall_ref, ncol_ref, nrow_ref, idx_ref):
    xr = xrows_ref[...]
    xa = xall_ref[...]
    inner = jax.lax.dot_general(xr, xa, (((1,), (1,)), ((), ())),
                                preferred_element_type=jnp.float32,
                                precision=jax.lax.Precision.DEFAULT)
    D = (2.0 * inner - ncol_ref[...]) - nrow_ref[...]
    idx_ref[...] = _file_topk(D)


@functools.partial(jax.jit, static_argnames=("tm",))
def _knn_idx(xT, nrm, tm=512):
    C = xT.shape[1]
    grid = (N // tm,)
    return pl.pallas_call(
        _knn_body,
        grid=grid,
        compiler_params=pltpu.CompilerParams(
            vmem_limit_bytes=100 * 1024 * 1024),
        in_specs=[
            pl.BlockSpec((tm, C), lambda i: (i, 0)),
            pl.BlockSpec((N, C), lambda i: (0, 0)),
            pl.BlockSpec((1, N), lambda i: (0, 0)),
            pl.BlockSpec((tm, 1), lambda i: (i, 0)),
        ],
        out_specs=pl.BlockSpec((tm, K), lambda i: (i, 0)),
        out_shape=jax.ShapeDtypeStruct((N, K), jnp.int32),
    )(xT, xT, nrm[None, :], nrm[:, None])


def _conv_body(feat_ref, w_ref, out_ref):
    f = feat_ref[...]
    w = w_ref[...]
    y = jax.lax.dot_general(f, w, (((1,), (1,)), ((), ())),
                            preferred_element_type=jnp.float32,
                            precision=jax.lax.Precision.DEFAULT)
    tm = y.shape[0] // K
    m = jnp.max(y.reshape(tm, K, y.shape[1]), axis=1)
    out_ref[...] = _act(m)


@functools.partial(jax.jit, static_argnames=("tm",))
def _conv_max(feat, W, tm=128):
    O = W.shape[0]
    C2 = W.shape[1]
    grid = (N // tm,)
    return pl.pallas_call(
        _conv_body,
        grid=grid,
        in_specs=[
            pl.BlockSpec((tm * K, C2), lambda i: (i, 0)),
            pl.BlockSpec((O, C2), lambda i: (0, 0)),
        ],
        out_specs=pl.BlockSpec((tm, O), lambda i: (i, 0)),
        out_shape=jax.ShapeDtypeStruct((N, O), jnp.float32),
    )(feat, W)


def _edge_layer(xT, W):
    C = xT.shape[1]
    pad = (-C) % 8
    xp = jnp.pad(xT, ((0, 0), (0, pad))) if pad else xT
    nrm = jnp.sum(xT * xT, axis=1)
    idx = _knn_idx(xp, nrm)
    nb = jnp.take(xT, idx.reshape(-1), axis=0)
    center = jnp.repeat(xT, K, axis=0)
    feat = jnp.concatenate([nb - center, center], axis=1)
    return _conv_max(feat, W)


def kernel(x, W1, W2, W3, W4, W5):
    xT = x[0].T
    x1 = _edge_layer(xT, W1)
    x2 = _edge_layer(x1, W2)
    x3 = _edge_layer(x2, W3)
    x4 = _edge_layer(x3, W4)
    cat = jnp.concatenate([x1, x2, x3, x4], axis=1)
    x5 = _act(cat @ W5.T)
    xg = jnp.max(x5, axis=0)
    x5t = x5.T
    out = jnp.concatenate(
        [x5t, jnp.broadcast_to(xg[:, None], x5t.shape)], axis=0)
    return out[None]


# speedup vs baseline: 3.1676x; 1.4215x over previous
import functools
import jax
import jax.numpy as jnp
from jax import lax
from jax.experimental import pallas as pl
from jax.experimental.pallas import tpu as pltpu
from jax.experimental.pallas import tpu_sc as plsc

N = 8192
K = 20
BN_EPS = 1e-5
NEG = -jnp.inf


def _leaky(y):
    return jnp.where(y >= 0, y, 0.2 * y)


def _act(y):
    return _leaky(y / jnp.sqrt(1.0 + BN_EPS))


CHUNK = 128
NCHUNK = N // CHUNK
DEPTH = 6


def _naive_topk(D):
    TM = D.shape[0]
    iota = jax.lax.broadcasted_iota(jnp.int32, (TM, N), 1)
    cols = []
    for _ in range(K):
        m = jnp.max(D, axis=1, keepdims=True)
        sel = jnp.min(jnp.where(D == m, iota, N), axis=1, keepdims=True)
        cols.append(sel)
        D = jnp.where(iota == sel, NEG, D)
    return jnp.concatenate(cols, axis=1)


def _chunked_topk(D):
    TM = D.shape[0]
    Dr = D.reshape(TM, NCHUNK, CHUNK)
    iota_s = jax.lax.broadcasted_iota(jnp.int32, (TM, NCHUNK, CHUNK), 2)
    Mv = []
    Pv = []
    m = jnp.max(Dr, axis=2)
    p = jnp.min(jnp.where(Dr == m[:, :, None], iota_s, CHUNK), axis=2)
    Mv.append(m)
    Pv.append(p)
    for _ in range(1, DEPTH):
        mp = Mv[-1][:, :, None]
        pp = Pv[-1][:, :, None]
        keep = (Dr < mp) | ((Dr == mp) & (iota_s > pp))
        m = jnp.max(jnp.where(keep, Dr, NEG), axis=2)
        p = jnp.min(jnp.where((Dr == m[:, :, None]) & keep, iota_s, CHUNK),
                    axis=2)
        Mv.append(m)
        Pv.append(p)

    iota_g = jax.lax.broadcasted_iota(jnp.int32, (TM, NCHUNK), 1)
    H = Mv[0]
    HP = Pv[0]
    CNT = jnp.ones((TM, NCHUNK), jnp.int32)
    bad = jnp.zeros((TM, 1), jnp.bool_)
    cols = []
    for _ in range(K):
        m = jnp.max(H, axis=1, keepdims=True)
        gsel = jnp.min(jnp.where(H == m, iota_g, NCHUNK), axis=1,
                       keepdims=True)
        hit = iota_g == gsel
        pos = jnp.sum(jnp.where(hit, HP, 0), axis=1, keepdims=True)
        cols.append(gsel * CHUNK + pos)
        nh = NEG * jnp.ones_like(H)
        np_ = jnp.full_like(HP, CHUNK)
        for d in range(1, DEPTH):
            sel_d = CNT == d
            nh = jnp.where(sel_d, Mv[d], nh)
            np_ = jnp.where(sel_d, Pv[d], np_)
        exhausted = CNT >= DEPTH
        bad = bad | jnp.any(hit & exhausted, axis=1, keepdims=True)
        H = jnp.where(hit, nh, H)
        HP = jnp.where(hit, np_, HP)
        CNT = jnp.where(hit, CNT + 1, CNT)
    fast = jnp.concatenate(cols, axis=1)
    return jax.lax.cond(jnp.any(bad), lambda: _naive_topk(D), lambda: fast)


def _argmax_topk(D):
    TM = D.shape[0]
    iota = jax.lax.broadcasted_iota(jnp.int32, (TM, N), 1)
    cols = []
    for _ in range(K):
        sel = jnp.argmax(D, axis=1)[:, None]
        cols.append(sel)
        D = jnp.where(iota == sel, NEG, D)
    return jnp.concatenate(cols, axis=1)


LANES = 128
GSTEPS = N // LANES
FDEPTH = 5


def _file_topk(D):
    TM = D.shape[0]
    vs = [jnp.full((TM, LANES), NEG, jnp.float32) for _ in range(FDEPTH)]
    ps = [jnp.zeros((TM, LANES), jnp.int32) for _ in range(FDEPTH)]
    for g in range(GSTEPS):
        x = jax.lax.slice(D, (0, g * LANES), (TM, (g + 1) * LANES))
        gi = jnp.full((TM, LANES), g, jnp.int32)
        for d in range(FDEPTH):
            c = x > vs[d]
            nv = jnp.maximum(vs[d], x)
            dv = jnp.minimum(vs[d], x)
            np_ = jnp.where(c, gi, ps[d])
            dp = jnp.where(c, ps[d], gi)
            vs[d] = nv
            ps[d] = np_
            x = dv
            gi = dp

    iota_l = jax.lax.broadcasted_iota(jnp.int32, (TM, LANES), 1)
    H = vs[0]
    HP = ps[0]
    CNT = jnp.ones((TM, LANES), jnp.int32)
    bad = jnp.zeros((TM, 1), jnp.bool_)
    cols = []
    for _ in range(K):
        m = jnp.max(H, axis=1, keepdims=True)
        ssel = jnp.min(jnp.where(H == m, iota_l, LANES), axis=1,
                       keepdims=True)
        hit = iota_l == ssel
        gpos = jnp.sum(jnp.where(hit, HP, 0), axis=1, keepdims=True)
        cols.append(gpos * LANES + ssel)
        nh = jnp.full_like(H, NEG)
        nhp = jnp.zeros_like(HP)
        for d in range(1, FDEPTH):
            sel_d = CNT == d
            nh = jnp.where(sel_d, vs[d], nh)
            nhp = jnp.where(sel_d, ps[d], nhp)
        bad = bad | jnp.any(hit & (CNT >= FDEPTH), axis=1, keepdims=True)
        H = jnp.where(hit, nh, H)
        HP = jnp.where(hit, nhp, HP)
        CNT = jnp.where(hit, CNT + 1, CNT)
    fast = jnp.concatenate(cols, axis=1)
    return jax.lax.cond(jnp.any(bad), lambda: _argmax_topk(D), lambda: fast)


def _knn_body(xrows_ref, xall_ref, ncol_ref, nrow_ref, idx_ref):
    xr = xrows_ref[...]
    xa = xall_ref[...]
    inner = jax.lax.dot_general(xr, xa, (((1,), (1,)), ((), ())),
                                preferred_element_type=jnp.float32,
                                precision=jax.lax.Precision.DEFAULT)
    D = (2.0 * inner - ncol_ref[...]) - nrow_ref[...]
    idx_ref[...] = _file_topk(D)


@functools.partial(jax.jit, static_argnames=("tm",))
def _knn_idx(xT, nrm, tm=512):
    C = xT.shape[1]
    grid = (N // tm,)
    return pl.pallas_call(
        _knn_body,
        grid=grid,
        compiler_params=pltpu.CompilerParams(
            vmem_limit_bytes=100 * 1024 * 1024),
        in_specs=[
            pl.BlockSpec((tm, C), lambda i: (i, 0)),
            pl.BlockSpec((N, C), lambda i: (0, 0)),
            pl.BlockSpec((1, N), lambda i: (0, 0)),
            pl.BlockSpec((tm, 1), lambda i: (i, 0)),
        ],
        out_specs=pl.BlockSpec((tm, K), lambda i: (i, 0)),
        out_shape=jax.ShapeDtypeStruct((N, K), jnp.int32),
    )(xT, xT, nrm[None, :], nrm[:, None])


def _conv_body(feat_ref, w_ref, out_ref):
    f = feat_ref[...]
    w = w_ref[...]
    y = jax.lax.dot_general(f, w, (((1,), (1,)), ((), ())),
                            preferred_element_type=jnp.float32,
                            precision=jax.lax.Precision.DEFAULT)
    tm = y.shape[0] // K
    m = jnp.max(y.reshape(tm, K, y.shape[1]), axis=1)
    out_ref[...] = _act(m)


@functools.partial(jax.jit, static_argnames=("tm",))
def _conv_max(feat, W, tm=128):
    O = W.shape[0]
    C2 = W.shape[1]
    grid = (N // tm,)
    return pl.pallas_call(
        _conv_body,
        grid=grid,
        in_specs=[
            pl.BlockSpec((tm * K, C2), lambda i: (i, 0)),
            pl.BlockSpec((O, C2), lambda i: (0, 0)),
        ],
        out_specs=pl.BlockSpec((tm, O), lambda i: (i, 0)),
        out_shape=jax.ShapeDtypeStruct((N, O), jnp.float32),
    )(feat, W)


NW = 32
PPW = N // NW
PB = 4
NB = PPW // PB


@functools.partial(jax.jit, static_argnames=("c",))
def _sc_gather_feat(xTp, idxf, c):
    cpad = xTp.shape[1]
    mesh = plsc.VectorSubcoreMesh(core_axis_name="c", subcore_axis_name="s")

    @functools.partial(
        pl.kernel, mesh=mesh,
        out_type=jax.ShapeDtypeStruct((N * K, 2 * c), jnp.float32),
        scratch_types=[
            pltpu.VMEM((PPW * K,), jnp.int32),
            pltpu.VMEM((PB * K, cpad), jnp.float32),
            pltpu.VMEM((PB, cpad), jnp.float32),
            pltpu.VMEM((PB * K, 2 * c), jnp.float32),
            pltpu.SemaphoreType.DMA,
        ],
    )
    def k(xT_hbm, idxf_hbm, out_hbm, idx_v, rows_v, ctr_v, feat_v, sem):
        wid = lax.axis_index("s") * 2 + lax.axis_index("c")
        base = wid * PPW
        pltpu.sync_copy(idxf_hbm.at[pl.ds(base * K, PPW * K)], idx_v)

        def body(b, carry):
            pltpu.async_copy(
                xT_hbm.at[idx_v.at[pl.ds(b * PB * K, PB * K)]],
                rows_v, sem).wait()
            pltpu.sync_copy(xT_hbm.at[pl.ds(base + b * PB, PB)], ctr_v)
            for p in range(PB):
                for cc in range(c // 16):
                    cv = ctr_v[p, pl.ds(cc * 16, 16)]
                    for j in range(K):
                        e = p * K + j
                        feat_v[e, pl.ds(cc * 16, 16)] = (
                            rows_v[e, pl.ds(cc * 16, 16)] - cv)
                        feat_v[e, pl.ds(c + cc * 16, 16)] = cv
            pltpu.sync_copy(
                feat_v, out_hbm.at[pl.ds((base + b * PB) * K, PB * K)])
            return carry

        lax.fori_loop(0, NB, body, 0)

    return k(xTp, idxf)


def _edge_layer(xT, W):
    C = xT.shape[1]
    pad = (-C) % 8
    xp = jnp.pad(xT, ((0, 0), (0, pad))) if pad else xT
    nrm = jnp.sum(xT * xT, axis=1)
    idx = _knn_idx(xp, nrm)
    if C % 16 == 0:
        cp = (-C) % 128
        xg = jnp.pad(xT, ((0, 0), (0, cp))) if cp else xT
        feat = _sc_gather_feat(xg, idx.reshape(-1), C)
    else:
        nb = jnp.take(xT, idx.reshape(-1), axis=0)
        center = jnp.repeat(xT, K, axis=0)
        feat = jnp.concatenate([nb - center, center], axis=1)
    return _conv_max(feat, W)


def kernel(x, W1, W2, W3, W4, W5):
    xT = x[0].T
    x1 = _edge_layer(xT, W1)
    x2 = _edge_layer(x1, W2)
    x3 = _edge_layer(x2, W3)
    x4 = _edge_layer(x3, W4)
    cat = jnp.concatenate([x1, x2, x3, x4], axis=1)
    x5 = _act(cat @ W5.T)
    xg = jnp.max(x5, axis=0)
    x5t = x5.T
    out = jnp.concatenate(
        [x5t, jnp.broadcast_to(xg[:, None], x5t.shape)], axis=0)
    return out[None]


# speedup vs baseline: 3.1886x; 1.0066x over previous
import functools
import jax
import jax.numpy as jnp
from jax import lax
from jax.experimental import pallas as pl
from jax.experimental.pallas import tpu as pltpu
from jax.experimental.pallas import tpu_sc as plsc

N = 8192
K = 20
BN_EPS = 1e-5
NEG = -jnp.inf


def _leaky(y):
    return jnp.where(y >= 0, y, 0.2 * y)


def _act(y):
    return _leaky(y / jnp.sqrt(1.0 + BN_EPS))


CHUNK = 128
NCHUNK = N // CHUNK
DEPTH = 6


def _naive_topk(D):
    TM = D.shape[0]
    iota = jax.lax.broadcasted_iota(jnp.int32, (TM, N), 1)
    cols = []
    for _ in range(K):
        m = jnp.max(D, axis=1, keepdims=True)
        sel = jnp.min(jnp.where(D == m, iota, N), axis=1, keepdims=True)
        cols.append(sel)
        D = jnp.where(iota == sel, NEG, D)
    return jnp.concatenate(cols, axis=1)


def _chunked_topk(D):
    TM = D.shape[0]
    Dr = D.reshape(TM, NCHUNK, CHUNK)
    iota_s = jax.lax.broadcasted_iota(jnp.int32, (TM, NCHUNK, CHUNK), 2)
    Mv = []
    Pv = []
    m = jnp.max(Dr, axis=2)
    p = jnp.min(jnp.where(Dr == m[:, :, None], iota_s, CHUNK), axis=2)
    Mv.append(m)
    Pv.append(p)
    for _ in range(1, DEPTH):
        mp = Mv[-1][:, :, None]
        pp = Pv[-1][:, :, None]
        keep = (Dr < mp) | ((Dr == mp) & (iota_s > pp))
        m = jnp.max(jnp.where(keep, Dr, NEG), axis=2)
        p = jnp.min(jnp.where((Dr == m[:, :, None]) & keep, iota_s, CHUNK),
                    axis=2)
        Mv.append(m)
        Pv.append(p)

    iota_g = jax.lax.broadcasted_iota(jnp.int32, (TM, NCHUNK), 1)
    H = Mv[0]
    HP = Pv[0]
    CNT = jnp.ones((TM, NCHUNK), jnp.int32)
    bad = jnp.zeros((TM, 1), jnp.bool_)
    cols = []
    for _ in range(K):
        m = jnp.max(H, axis=1, keepdims=True)
        gsel = jnp.min(jnp.where(H == m, iota_g, NCHUNK), axis=1,
                       keepdims=True)
        hit = iota_g == gsel
        pos = jnp.sum(jnp.where(hit, HP, 0), axis=1, keepdims=True)
        cols.append(gsel * CHUNK + pos)
        nh = NEG * jnp.ones_like(H)
        np_ = jnp.full_like(HP, CHUNK)
        for d in range(1, DEPTH):
            sel_d = CNT == d
            nh = jnp.where(sel_d, Mv[d], nh)
            np_ = jnp.where(sel_d, Pv[d], np_)
        exhausted = CNT >= DEPTH
        bad = bad | jnp.any(hit & exhausted, axis=1, keepdims=True)
        H = jnp.where(hit, nh, H)
        HP = jnp.where(hit, np_, HP)
        CNT = jnp.where(hit, CNT + 1, CNT)
    fast = jnp.concatenate(cols, axis=1)
    return jax.lax.cond(jnp.any(bad), lambda: _naive_topk(D), lambda: fast)


def _argmax_topk(D):
    TM = D.shape[0]
    iota = jax.lax.broadcasted_iota(jnp.int32, (TM, N), 1)
    cols = []
    for _ in range(K):
        sel = jnp.argmax(D, axis=1)[:, None]
        cols.append(sel)
        D = jnp.where(iota == sel, NEG, D)
    return jnp.concatenate(cols, axis=1)


LANES = 128
GSTEPS = N // LANES
FDEPTH = 5


def _file_topk(D):
    TM = D.shape[0]
    vs = [jnp.full((TM, LANES), NEG, jnp.float32) for _ in range(FDEPTH)]
    ps = [jnp.zeros((TM, LANES), jnp.int32) for _ in range(FDEPTH)]
    for g in range(GSTEPS):
        x = jax.lax.slice(D, (0, g * LANES), (TM, (g + 1) * LANES))
        gi = jnp.full((TM, LANES), g, jnp.int32)
        for d in range(FDEPTH):
            c = x > vs[d]
            nv = jnp.maximum(vs[d], x)
            dv = jnp.minimum(vs[d], x)
            np_ = jnp.where(c, gi, ps[d])
            dp = jnp.where(c, ps[d], gi)
            vs[d] = nv
            ps[d] = np_
            x = dv
            gi = dp

    iota_l = jax.lax.broadcasted_iota(jnp.int32, (TM, LANES), 1)
    H = vs[0]
    HP = ps[0]
    CNT = jnp.ones((TM, LANES), jnp.int32)
    bad = jnp.zeros((TM, 1), jnp.bool_)
    cols = []
    for _ in range(K):
        m = jnp.max(H, axis=1, keepdims=True)
        ssel = jnp.min(jnp.where(H == m, iota_l, LANES), axis=1,
                       keepdims=True)
        hit = iota_l == ssel
        gpos = jnp.sum(jnp.where(hit, HP, 0), axis=1, keepdims=True)
        cols.append(gpos * LANES + ssel)
        nh = jnp.full_like(H, NEG)
        nhp = jnp.zeros_like(HP)
        for d in range(1, FDEPTH):
            sel_d = CNT == d
            nh = jnp.where(sel_d, vs[d], nh)
            nhp = jnp.where(sel_d, ps[d], nhp)
        bad = bad | jnp.any(hit & (CNT >= FDEPTH), axis=1, keepdims=True)
        H = jnp.where(hit, nh, H)
        HP = jnp.where(hit, nhp, HP)
        CNT = jnp.where(hit, CNT + 1, CNT)
    fast = jnp.concatenate(cols, axis=1)
    return jax.lax.cond(jnp.any(bad), lambda: _argmax_topk(D), lambda: fast)


def _knn_body(xrows_ref, xall_ref, ncol_ref, nrow_ref, idx_ref):
    xr = xrows_ref[...]
    xa = xall_ref[...]
    inner = jax.lax.dot_general(xr, xa, (((1,), (1,)), ((), ())),
                                preferred_element_type=jnp.float32,
                                precision=jax.lax.Precision.DEFAULT)
    D = (2.0 * inner - ncol_ref[...]) - nrow_ref[...]
    idx_ref[...] = _file_topk(D)


@functools.partial(jax.jit, static_argnames=("tm",))
def _knn_idx(xT, nrm, tm=512):
    C = xT.shape[1]
    grid = (N // tm,)
    return pl.pallas_call(
        _knn_body,
        grid=grid,
        compiler_params=pltpu.CompilerParams(
            vmem_limit_bytes=100 * 1024 * 1024),
        in_specs=[
            pl.BlockSpec((tm, C), lambda i: (i, 0)),
            pl.BlockSpec((N, C), lambda i: (0, 0)),
            pl.BlockSpec((1, N), lambda i: (0, 0)),
            pl.BlockSpec((tm, 1), lambda i: (i, 0)),
        ],
        out_specs=pl.BlockSpec((tm, K), lambda i: (i, 0)),
        out_shape=jax.ShapeDtypeStruct((N, K), jnp.int32),
    )(xT, xT, nrm[None, :], nrm[:, None])


def _conv_body(feat_ref, w_ref, out_ref):
    f = feat_ref[...]
    w = w_ref[...]
    y = jax.lax.dot_general(f, w, (((1,), (1,)), ((), ())),
                            preferred_element_type=jnp.float32,
                            precision=jax.lax.Precision.DEFAULT)
    tm = y.shape[0] // K
    m = jnp.max(y.reshape(tm, K, y.shape[1]), axis=1)
    out_ref[...] = _act(m)


@functools.partial(jax.jit, static_argnames=("tm",))
def _conv_max(feat, W, tm=128):
    O = W.shape[0]
    C2 = W.shape[1]
    grid = (N // tm,)
    return pl.pallas_call(
        _conv_body,
        grid=grid,
        in_specs=[
            pl.BlockSpec((tm * K, C2), lambda i: (i, 0)),
            pl.BlockSpec((O, C2), lambda i: (0, 0)),
        ],
        out_specs=pl.BlockSpec((tm, O), lambda i: (i, 0)),
        out_shape=jax.ShapeDtypeStruct((N, O), jnp.float32),
    )(feat, W)


NW = 32
PPW = N // NW
PB = 4
NB = PPW // PB


@functools.partial(jax.jit, static_argnames=("c",))
def _sc_gather_feat(xTp, idxf, c):
    cpad = xTp.shape[1]
    mesh = plsc.VectorSubcoreMesh(core_axis_name="c", subcore_axis_name="s")

    @functools.partial(
        pl.kernel, mesh=mesh,
        out_type=jax.ShapeDtypeStruct((N * K, 2 * c), jnp.float32),
        scratch_types=[
            pltpu.VMEM((PPW * K,), jnp.int32),
            pltpu.VMEM((PB * K, cpad), jnp.float32),
            pltpu.VMEM((PB, cpad), jnp.float32),
            pltpu.VMEM((PB * K, 2 * c), jnp.float32),
            pltpu.SemaphoreType.DMA,
        ],
    )
    def k(xT_hbm, idxf_hbm, out_hbm, idx_v, rows_v, ctr_v, feat_v, sem):
        wid = lax.axis_index("s") * 2 + lax.axis_index("c")
        base = wid * PPW
        pltpu.sync_copy(idxf_hbm.at[pl.ds(base * K, PPW * K)], idx_v)

        def body(b, carry):
            pltpu.async_copy(
                xT_hbm.at[idx_v.at[pl.ds(b * PB * K, PB * K)]],
                rows_v, sem).wait()
            pltpu.sync_copy(xT_hbm.at[pl.ds(base + b * PB, PB)], ctr_v)
            for p in range(PB):
                for cc in range(c // 16):
                    cv = ctr_v[p, pl.ds(cc * 16, 16)]
                    for j in range(K):
                        e = p * K + j
                        feat_v[e, pl.ds(cc * 16, 16)] = (
                            rows_v[e, pl.ds(cc * 16, 16)] - cv)
                        feat_v[e, pl.ds(c + cc * 16, 16)] = cv
            pltpu.sync_copy(
                feat_v, out_hbm.at[pl.ds((base + b * PB) * K, PB * K)])
            return carry

        lax.fori_loop(0, NB, body, 0)

    return k(xTp, idxf)


def _edge_layer(xT, W):
    C = xT.shape[1]
    pad = (-C) % 8
    xp = jnp.pad(xT, ((0, 0), (0, pad))) if pad else xT
    nrm = jnp.sum(xT * xT, axis=1)
    idx = _knn_idx(xp, nrm)
    if C % 16 == 0:
        cp = (-C) % 128
        xg = jnp.pad(xT, ((0, 0), (0, cp))) if cp else xT
        feat = _sc_gather_feat(xg, idx.reshape(-1), C)
    else:
        nb = jnp.take(xT, idx.reshape(-1), axis=0)
        center = jnp.repeat(xT, K, axis=0)
        feat = jnp.concatenate([nb - center, center], axis=1)
    return _conv_max(feat, W)


def _final_body(x1_ref, x2_ref, x3_ref, x4_ref, w_ref, x5t_ref, xg_ref):
    i = pl.program_id(0)
    c = jnp.concatenate(
        [x1_ref[...], x2_ref[...], x3_ref[...], x4_ref[...]], axis=1)
    y = jax.lax.dot_general(w_ref[...], c, (((1,), (1,)), ((), ())),
                            preferred_element_type=jnp.float32,
                            precision=jax.lax.Precision.DEFAULT)
    y = _act(y)
    x5t_ref[...] = y
    m = jnp.max(y, axis=1, keepdims=True)

    @pl.when(i == 0)
    def _():
        xg_ref[...] = m

    @pl.when(i > 0)
    def _():
        xg_ref[...] = jnp.maximum(xg_ref[...], m)


@functools.partial(jax.jit, static_argnames=("tn",))
def _final(x1, x2, x3, x4, W5, tn=1024):
    grid = (N // tn,)
    return pl.pallas_call(
        _final_body,
        grid=grid,
        in_specs=[
            pl.BlockSpec((tn, 64), lambda i: (i, 0)),
            pl.BlockSpec((tn, 64), lambda i: (i, 0)),
            pl.BlockSpec((tn, 128), lambda i: (i, 0)),
            pl.BlockSpec((tn, 256), lambda i: (i, 0)),
            pl.BlockSpec((512, 512), lambda i: (0, 0)),
        ],
        out_specs=[
            pl.BlockSpec((512, tn), lambda i: (0, i)),
            pl.BlockSpec((512, 1), lambda i: (0, 0)),
        ],
        out_shape=[
            jax.ShapeDtypeStruct((512, N), jnp.float32),
            jax.ShapeDtypeStruct((512, 1), jnp.float32),
        ],
    )(x1, x2, x3, x4, W5)


def kernel(x, W1, W2, W3, W4, W5):
    xT = x[0].T
    x1 = _edge_layer(xT, W1)
    x2 = _edge_layer(x1, W2)
    x3 = _edge_layer(x2, W3)
    x4 = _edge_layer(x3, W4)
    x5t, xg = _final(x1, x2, x3, x4, W5)
    out = jnp.concatenate(
        [x5t, jnp.broadcast_to(xg, x5t.shape)], axis=0)
    return out[None]


# speedup vs baseline: 3.7176x; 1.1659x over previous
import functools
import jax
import jax.numpy as jnp
from jax import lax
from jax.experimental import pallas as pl
from jax.experimental.pallas import tpu as pltpu
from jax.experimental.pallas import tpu_sc as plsc

N = 8192
K = 20
BN_EPS = 1e-5
NEG = -jnp.inf


def _leaky(y):
    return jnp.where(y >= 0, y, 0.2 * y)


def _act(y):
    return _leaky(y / jnp.sqrt(1.0 + BN_EPS))


CHUNK = 128
NCHUNK = N // CHUNK
DEPTH = 6


def _naive_topk(D):
    TM = D.shape[0]
    iota = jax.lax.broadcasted_iota(jnp.int32, (TM, N), 1)
    cols = []
    for _ in range(K):
        m = jnp.max(D, axis=1, keepdims=True)
        sel = jnp.min(jnp.where(D == m, iota, N), axis=1, keepdims=True)
        cols.append(sel)
        D = jnp.where(iota == sel, NEG, D)
    return jnp.concatenate(cols, axis=1)


def _chunked_topk(D):
    TM = D.shape[0]
    Dr = D.reshape(TM, NCHUNK, CHUNK)
    iota_s = jax.lax.broadcasted_iota(jnp.int32, (TM, NCHUNK, CHUNK), 2)
    Mv = []
    Pv = []
    m = jnp.max(Dr, axis=2)
    p = jnp.min(jnp.where(Dr == m[:, :, None], iota_s, CHUNK), axis=2)
    Mv.append(m)
    Pv.append(p)
    for _ in range(1, DEPTH):
        mp = Mv[-1][:, :, None]
        pp = Pv[-1][:, :, None]
        keep = (Dr < mp) | ((Dr == mp) & (iota_s > pp))
        m = jnp.max(jnp.where(keep, Dr, NEG), axis=2)
        p = jnp.min(jnp.where((Dr == m[:, :, None]) & keep, iota_s, CHUNK),
                    axis=2)
        Mv.append(m)
        Pv.append(p)

    iota_g = jax.lax.broadcasted_iota(jnp.int32, (TM, NCHUNK), 1)
    H = Mv[0]
    HP = Pv[0]
    CNT = jnp.ones((TM, NCHUNK), jnp.int32)
    bad = jnp.zeros((TM, 1), jnp.bool_)
    cols = []
    for _ in range(K):
        m = jnp.max(H, axis=1, keepdims=True)
        gsel = jnp.min(jnp.where(H == m, iota_g, NCHUNK), axis=1,
                       keepdims=True)
        hit = iota_g == gsel
        pos = jnp.sum(jnp.where(hit, HP, 0), axis=1, keepdims=True)
        cols.append(gsel * CHUNK + pos)
        nh = NEG * jnp.ones_like(H)
        np_ = jnp.full_like(HP, CHUNK)
        for d in range(1, DEPTH):
            sel_d = CNT == d
            nh = jnp.where(sel_d, Mv[d], nh)
            np_ = jnp.where(sel_d, Pv[d], np_)
        exhausted = CNT >= DEPTH
        bad = bad | jnp.any(hit & exhausted, axis=1, keepdims=True)
        H = jnp.where(hit, nh, H)
        HP = jnp.where(hit, np_, HP)
        CNT = jnp.where(hit, CNT + 1, CNT)
    fast = jnp.concatenate(cols, axis=1)
    return jax.lax.cond(jnp.any(bad), lambda: _naive_topk(D), lambda: fast)


def _argmax_topk(D):
    TM = D.shape[0]
    iota = jax.lax.broadcasted_iota(jnp.int32, (TM, N), 1)
    cols = []
    for _ in range(K):
        sel = jnp.argmax(D, axis=1)[:, None]
        cols.append(sel)
        D = jnp.where(iota == sel, NEG, D)
    return jnp.concatenate(cols, axis=1)


LANES = 128
GSTEPS = N // LANES
FDEPTH = 5


def _file_topk(D):
    TM = D.shape[0]
    vs = [jnp.full((TM, LANES), NEG, jnp.float32) for _ in range(FDEPTH)]
    ps = [jnp.zeros((TM, LANES), jnp.int32) for _ in range(FDEPTH)]
    for g in range(GSTEPS):
        x = jax.lax.slice(D, (0, g * LANES), (TM, (g + 1) * LANES))
        gi = jnp.full((TM, LANES), g, jnp.int32)
        for d in range(FDEPTH):
            c = x > vs[d]
            nv = jnp.maximum(vs[d], x)
            dv = jnp.minimum(vs[d], x)
            np_ = jnp.where(c, gi, ps[d])
            dp = jnp.where(c, ps[d], gi)
            vs[d] = nv
            ps[d] = np_
            x = dv
            gi = dp

    iota_l = jax.lax.broadcasted_iota(jnp.int32, (TM, LANES), 1)
    H = vs[0]
    HP = ps[0]
    CNT = jnp.ones((TM, LANES), jnp.int32)
    bad = jnp.zeros((TM, 1), jnp.bool_)
    cols = []
    for _ in range(K):
        m = jnp.max(H, axis=1, keepdims=True)
        ssel = jnp.min(jnp.where(H == m, iota_l, LANES), axis=1,
                       keepdims=True)
        hit = iota_l == ssel
        gpos = jnp.sum(jnp.where(hit, HP, 0), axis=1, keepdims=True)
        cols.append(gpos * LANES + ssel)
        nh = jnp.full_like(H, NEG)
        nhp = jnp.zeros_like(HP)
        for d in range(1, FDEPTH):
            sel_d = CNT == d
            nh = jnp.where(sel_d, vs[d], nh)
            nhp = jnp.where(sel_d, ps[d], nhp)
        bad = bad | jnp.any(hit & (CNT >= FDEPTH), axis=1, keepdims=True)
        H = jnp.where(hit, nh, H)
        HP = jnp.where(hit, nhp, HP)
        CNT = jnp.where(hit, CNT + 1, CNT)
    fast = jnp.concatenate(cols, axis=1)
    return jax.lax.cond(jnp.any(bad), lambda: _argmax_topk(D), lambda: fast)


def _knn_body(xrows_ref, xall_ref, ncol_ref, nrow_ref, idx_ref):
    xr = xrows_ref[...]
    xa = xall_ref[...]
    inner = jax.lax.dot_general(xr, xa, (((1,), (1,)), ((), ())),
                                preferred_element_type=jnp.float32,
                                precision=jax.lax.Precision.DEFAULT)
    D = (2.0 * inner - ncol_ref[...]) - nrow_ref[...]
    idx_ref[...] = _file_topk(D)


@functools.partial(jax.jit, static_argnames=("tm",))
def _knn_idx(xT, nrm, tm=512):
    C = xT.shape[1]
    grid = (N // tm,)
    return pl.pallas_call(
        _knn_body,
        grid=grid,
        compiler_params=pltpu.CompilerParams(
            vmem_limit_bytes=100 * 1024 * 1024),
        in_specs=[
            pl.BlockSpec((tm, C), lambda i: (i, 0)),
            pl.BlockSpec((N, C), lambda i: (0, 0)),
            pl.BlockSpec((1, N), lambda i: (0, 0)),
            pl.BlockSpec((tm, 1), lambda i: (i, 0)),
        ],
        out_specs=pl.BlockSpec((tm, K), lambda i: (i, 0)),
        out_shape=jax.ShapeDtypeStruct((N, K), jnp.int32),
    )(xT, xT, nrm[None, :], nrm[:, None])


def _conv_body(feat_ref, w_ref, out_ref):
    f = feat_ref[...]
    w = w_ref[...]
    y = jax.lax.dot_general(f, w, (((1,), (1,)), ((), ())),
                            preferred_element_type=jnp.float32,
                            precision=jax.lax.Precision.DEFAULT)
    tm = y.shape[0] // K
    m = jnp.max(y.reshape(tm, K, y.shape[1]), axis=1)
    out_ref[...] = _act(m)


@functools.partial(jax.jit, static_argnames=("tm",))
def _conv_max(feat, W, tm=128):
    O = W.shape[0]
    C2 = W.shape[1]
    grid = (N // tm,)
    return pl.pallas_call(
        _conv_body,
        grid=grid,
        in_specs=[
            pl.BlockSpec((tm * K, C2), lambda i: (i, 0)),
            pl.BlockSpec((O, C2), lambda i: (0, 0)),
        ],
        out_specs=pl.BlockSpec((tm, O), lambda i: (i, 0)),
        out_shape=jax.ShapeDtypeStruct((N, O), jnp.float32),
    )(feat, W)


NW = 32
PPW = N // NW
PB = 4
NB = PPW // PB


@functools.partial(jax.jit, static_argnames=("c",))
def _sc_gather_feat(xTp, idxf, c):
    cpad = xTp.shape[1]
    mesh = plsc.VectorSubcoreMesh(core_axis_name="c", subcore_axis_name="s")

    @functools.partial(
        pl.kernel, mesh=mesh,
        out_type=jax.ShapeDtypeStruct((N * K, 2 * c), jnp.float32),
        scratch_types=[
            pltpu.VMEM((PPW * K,), jnp.int32),
            pltpu.VMEM((PB * K, cpad), jnp.float32),
            pltpu.VMEM((PB, cpad), jnp.float32),
            pltpu.VMEM((PB * K, 2 * c), jnp.float32),
            pltpu.SemaphoreType.DMA,
        ],
    )
    def k(xT_hbm, idxf_hbm, out_hbm, idx_v, rows_v, ctr_v, feat_v, sem):
        wid = lax.axis_index("s") * 2 + lax.axis_index("c")
        base = wid * PPW
        pltpu.sync_copy(idxf_hbm.at[pl.ds(base * K, PPW * K)], idx_v)

        def body(b, carry):
            pltpu.async_copy(
                xT_hbm.at[idx_v.at[pl.ds(b * PB * K, PB * K)]],
                rows_v, sem).wait()
            pltpu.sync_copy(xT_hbm.at[pl.ds(base + b * PB, PB)], ctr_v)
            for p in range(PB):
                for cc in range(c // 16):
                    cv = ctr_v[p, pl.ds(cc * 16, 16)]
                    for j in range(K):
                        e = p * K + j
                        feat_v[e, pl.ds(cc * 16, 16)] = (
                            rows_v[e, pl.ds(cc * 16, 16)] - cv)
                        feat_v[e, pl.ds(c + cc * 16, 16)] = cv
            pltpu.sync_copy(
                feat_v, out_hbm.at[pl.ds((base + b * PB) * K, PB * K)])
            return carry

        lax.fori_loop(0, NB, body, 0)

    return k(xTp, idxf)


def _edge_layer(xT, W):
    C = xT.shape[1]
    pad = (-C) % 8
    xp = jnp.pad(xT, ((0, 0), (0, pad))) if pad else xT
    nrm = jnp.sum(xT * xT, axis=1)
    idx = _knn_idx(xp, nrm)
    Wg = W
    Cg = C
    if C % 16:
        Cg = C + ((-C) % 16)
        O = W.shape[0]
        Wg = jnp.zeros((O, 2 * Cg), W.dtype)
        Wg = Wg.at[:, :C].set(W[:, :C]).at[:, Cg:Cg + C].set(W[:, C:])
    cp = (-Cg) % 128
    xg = jnp.pad(xT, ((0, 0), (0, cp + Cg - C)))
    feat = _sc_gather_feat(xg, idx.reshape(-1), Cg)
    return _conv_max(feat, Wg)


def _final_body(x1_ref, x2_ref, x3_ref, x4_ref, w_ref, x5t_ref, xg_ref):
    i = pl.program_id(0)
    c = jnp.concatenate(
        [x1_ref[...], x2_ref[...], x3_ref[...], x4_ref[...]], axis=1)
    y = jax.lax.dot_general(w_ref[...], c, (((1,), (1,)), ((), ())),
                            preferred_element_type=jnp.float32,
                            precision=jax.lax.Precision.DEFAULT)
    y = _act(y)
    x5t_ref[...] = y
    m = jnp.max(y, axis=1, keepdims=True)

    @pl.when(i == 0)
    def _():
        xg_ref[...] = m

    @pl.when(i > 0)
    def _():
        xg_ref[...] = jnp.maximum(xg_ref[...], m)


@functools.partial(jax.jit, static_argnames=("tn",))
def _final(x1, x2, x3, x4, W5, tn=1024):
    grid = (N // tn,)
    return pl.pallas_call(
        _final_body,
        grid=grid,
        in_specs=[
            pl.BlockSpec((tn, 64), lambda i: (i, 0)),
            pl.BlockSpec((tn, 64), lambda i: (i, 0)),
            pl.BlockSpec((tn, 128), lambda i: (i, 0)),
            pl.BlockSpec((tn, 256), lambda i: (i, 0)),
            pl.BlockSpec((512, 512), lambda i: (0, 0)),
        ],
        out_specs=[
            pl.BlockSpec((512, tn), lambda i: (0, i)),
            pl.BlockSpec((512, 1), lambda i: (0, 0)),
        ],
        out_shape=[
            jax.ShapeDtypeStruct((512, N), jnp.float32),
            jax.ShapeDtypeStruct((512, 1), jnp.float32),
        ],
    )(x1, x2, x3, x4, W5)


def kernel(x, W1, W2, W3, W4, W5):
    xT = x[0].T
    x1 = _edge_layer(xT, W1)
    x2 = _edge_layer(x1, W2)
    x3 = _edge_layer(x2, W3)
    x4 = _edge_layer(x3, W4)
    x5t, xg = _final(x1, x2, x3, x4, W5)
    out = jnp.concatenate(
        [x5t, jnp.broadcast_to(xg, x5t.shape)], axis=0)
    return out[None]
